# baseline (speedup 1.0000x reference)
import functools

import jax
import jax.numpy as jnp
from jax import lax
from jax.experimental import pallas as pl
from jax.experimental.pallas import tpu as pltpu
from jax.experimental.pallas import tpu_sc as plsc

_V = 1000000
_B = 16384
_H = 200
_D = 32
_NC = 2
_NS = 16
_NW = _NC * _NS
_BPW = _B // _NW
_CROWS = 16
_NCHUNK = _BPW // _CROWS
_TPC = _CROWS * _H
_TOK_PER_W = _BPW * _H
_STAGE_TOKS = 25600
_PBLK = 98304


def _proj_body(xt_ref, w_ref, o_ref):
  res = lax.dot_general(w_ref[...], xt_ref[...], (((1,), (0,)), ((), ())),
                        preferred_element_type=jnp.float32)
  u0 = lax.convert_element_type(
      lax.bitcast_convert_type(res[0].astype(jnp.bfloat16), jnp.uint16),
      jnp.uint32)
  u1 = lax.convert_element_type(
      lax.bitcast_convert_type(res[1].astype(jnp.bfloat16), jnp.uint16),
      jnp.uint32)
  o_ref[...] = lax.bitcast_convert_type(u0 | (u1 << 16), jnp.int32)


_project = pl.pallas_call(
    _proj_body,
    grid=(pl.cdiv(_V, _PBLK),),
    in_specs=[
        pl.BlockSpec((_D, _PBLK), lambda i: (0, i)),
        pl.BlockSpec((2, _D), lambda i: (0, 0)),
    ],
    out_specs=pl.BlockSpec((_PBLK,), lambda i: (i,)),
    out_shape=jax.ShapeDtypeStruct((_V,), jnp.int32),
)


def _make_sc_kernel():
  mesh = plsc.VectorSubcoreMesh(core_axis_name="c", subcore_axis_name="s")

  @functools.partial(
      pl.kernel,
      mesh=mesh,
      out_type=jax.ShapeDtypeStruct((_B * 2,), jnp.float32),
      scratch_types=[
          pltpu.VMEM((_STAGE_TOKS,), jnp.int32),
          pltpu.VMEM((_TPC,), jnp.int32),
          pltpu.VMEM((_TPC,), jnp.int32),
          pltpu.VMEM((16,), jnp.float32),
          pltpu.VMEM((2 * _BPW,), jnp.float32),
          pltpu.SemaphoreType.DMA,
          pltpu.SemaphoreType.DMA,
      ],
      compiler_params=pltpu.CompilerParams(
          needs_layout_passes=False, use_tc_tiling_on_sc=False),
  )
  def k(tok_hbm, pp_hbm, b_hbm, out_hbm, idx_v, rv_a, rv_b, b_v, out_v,
        sem_a, sem_b):
    cid = lax.axis_index("c")
    sid = lax.axis_index("s")
    wid = sid * _NC + cid

    pltpu.sync_copy(b_hbm, b_v)
    bvec = b_v[pl.ds(0, 16)]
    bias0 = bvec[0]
    bias1 = bvec[1]
    lane = lax.iota(jnp.int32, 16)
    inv_h = jnp.float32(1.0 / _H)

    def stage(gg):
      t0 = wid * _TOK_PER_W + gg * _STAGE_TOKS
      pltpu.sync_copy(tok_hbm.at[pl.ds(t0, _STAGE_TOKS)], idx_v)

    def fire(g, rv, sem):
      g2 = g % 8
      idxs = idx_v.at[pl.ds(g2 * _TPC, _TPC)]
      pltpu.async_copy(pp_hbm.at[idxs], rv, sem)

    def drain(g, rv, sem):
      g2 = g % 8
      idxs = idx_v.at[pl.ds(g2 * _TPC, _TPC)]
      pltpu.make_async_copy(pp_hbm.at[idxs], rv, sem).wait()

    def unpack2(pw):
      bf = plsc.bitcast(pw, jnp.bfloat16)
      return plsc.unpack(bf, format=plsc.PackFormat.INTERLEAVED)

    def compute(g, rv):
      ov0 = jnp.zeros((16,), jnp.float32)
      ov1 = jnp.zeros((16,), jnp.float32)
      for b in range(_CROWS):
        z = jnp.zeros((16,), jnp.float32)
        base = b * _H

        def cstep(i, c):
          a0, a1 = c
          x0, x1 = unpack2(rv[pl.ds(base + i * 16, 16)])
          return a0 + x0, a1 + x1

        a0, a1 = lax.fori_loop(0, _H // 16, cstep, (z, z))
        t0v, t1v = unpack2(rv[pl.ds(base + _H - 16, 16)])
        a0 = a0 + jnp.where(lane >= 8, t0v, 0.0)
        a1 = a1 + jnp.where(lane >= 8, t1v, 0.0)
        o0 = jnp.sum(a0) * inv_h + bias0
        o1 = jnp.sum(a1) * inv_h + bias1
        ov0 = jnp.where(lane == b, o0, ov0)
        ov1 = jnp.where(lane == b, o1, ov1)
      out_v[pl.ds(g * _CROWS, 16)] = ov0
      out_v[pl.ds(_BPW + g * _CROWS, 16)] = ov1

    stage(0)
    fire(jnp.int32(0), rv_a, sem_a)

    def body(h, c):
      g0 = 2 * h
      g1 = 2 * h + 1
      drain(g0, rv_a, sem_a)
      fire(g1, rv_b, sem_b)
      compute(g0, rv_a)
      drain(g1, rv_b, sem_b)
      gn = g1 + 1

      @pl.when(jnp.logical_and(gn < _NCHUNK, gn % 8 == 0))
      def _():
        stage(gn // 8)

      @pl.when(gn < _NCHUNK)
      def _():
        fire(gn, rv_a, sem_a)

      compute(g1, rv_b)
      return c

    lax.fori_loop(0, _NCHUNK // 2, body, 0)
    pltpu.sync_copy(out_v.at[pl.ds(0, _BPW)],
                    out_hbm.at[pl.ds(wid * _BPW, _BPW)])
    pltpu.sync_copy(out_v.at[pl.ds(_BPW, _BPW)],
                    out_hbm.at[pl.ds(_B + wid * _BPW, _BPW)])

  return k


_sc_kernel = _make_sc_kernel()


@jax.jit
def kernel(token_index, emb_table, fc_w, fc_b):
  tok = token_index.astype(jnp.int32).reshape(-1)
  b_pad = jnp.zeros((16,), jnp.float32).at[:2].set(fc_b)
  pp = _project(emb_table.T, fc_w)
  out_flat = _sc_kernel(tok, pp, b_pad)
  return out_flat.reshape(2, _B).T


# speedup vs baseline: 1.0051x; 1.0051x over previous
import functools

import jax
import jax.numpy as jnp
from jax import lax
from jax.experimental import pallas as pl
from jax.experimental.pallas import tpu as pltpu
from jax.experimental.pallas import tpu_sc as plsc

_V = 1000000
_B = 16384
_H = 200
_D = 32
_NC = 2
_NS = 16
_NW = _NC * _NS
_BPW = _B // _NW
_CROWS = 16
_NCHUNK = _BPW // _CROWS
_TPC = _CROWS * _H
_TOK_PER_W = _BPW * _H
_STAGE_TOKS = 25600
_PBLK = 65536


def _proj_body(xt_ref, w_ref, o_ref):
  res = lax.dot_general(w_ref[...], xt_ref[...], (((1,), (0,)), ((), ())),
                        preferred_element_type=jnp.float32)
  u0 = lax.convert_element_type(
      lax.bitcast_convert_type(res[0].astype(jnp.bfloat16), jnp.uint16),
      jnp.uint32)
  u1 = lax.convert_element_type(
      lax.bitcast_convert_type(res[1].astype(jnp.bfloat16), jnp.uint16),
      jnp.uint32)
  o_ref[...] = lax.bitcast_convert_type(u0 | (u1 << 16), jnp.int32)


_project = pl.pallas_call(
    _proj_body,
    grid=(pl.cdiv(_V, _PBLK),),
    in_specs=[
        pl.BlockSpec((_D, _PBLK), lambda i: (0, i)),
        pl.BlockSpec((2, _D), lambda i: (0, 0)),
    ],
    out_specs=pl.BlockSpec((_PBLK,), lambda i: (i,)),
    out_shape=jax.ShapeDtypeStruct((_V,), jnp.int32),
)


def _make_sc_kernel():
  mesh = plsc.VectorSubcoreMesh(core_axis_name="c", subcore_axis_name="s")

  @functools.partial(
      pl.kernel,
      mesh=mesh,
      out_type=jax.ShapeDtypeStruct((_B * 2,), jnp.float32),
      scratch_types=[
          pltpu.VMEM((_STAGE_TOKS,), jnp.int32),
          pltpu.VMEM((_TPC,), jnp.int32),
          pltpu.VMEM((_TPC,), jnp.int32),
          pltpu.VMEM((16,), jnp.float32),
          pltpu.VMEM((2 * _BPW,), jnp.float32),
          pltpu.SemaphoreType.DMA,
          pltpu.SemaphoreType.DMA,
      ],
      compiler_params=pltpu.CompilerParams(
          needs_layout_passes=False, use_tc_tiling_on_sc=False),
  )
  def k(tok_hbm, pp_hbm, b_hbm, out_hbm, idx_v, rv_a, rv_b, b_v, out_v,
        sem_a, sem_b):
    cid = lax.axis_index("c")
    sid = lax.axis_index("s")
    wid = sid * _NC + cid

    pltpu.sync_copy(b_hbm, b_v)
    bvec = b_v[pl.ds(0, 16)]
    bias0 = bvec[0]
    bias1 = bvec[1]
    lane = lax.iota(jnp.int32, 16)
    inv_h = jnp.float32(1.0 / _H)

    def stage(gg):
      t0 = wid * _TOK_PER_W + gg * _STAGE_TOKS
      pltpu.sync_copy(tok_hbm.at[pl.ds(t0, _STAGE_TOKS)], idx_v)

    def fire(g, rv, sem):
      g2 = g % 8
      idxs = idx_v.at[pl.ds(g2 * _TPC, _TPC)]
      pltpu.async_copy(pp_hbm.at[idxs], rv, sem)

    def drain(g, rv, sem):
      g2 = g % 8
      idxs = idx_v.at[pl.ds(g2 * _TPC, _TPC)]
      pltpu.make_async_copy(pp_hbm.at[idxs], rv, sem).wait()

    def unpack2(pw):
      bf = plsc.bitcast(pw, jnp.bfloat16)
      return plsc.unpack(bf, format=plsc.PackFormat.INTERLEAVED)

    def compute(g, rv):
      ov0 = jnp.zeros((16,), jnp.float32)
      ov1 = jnp.zeros((16,), jnp.float32)
      for b in range(_CROWS):
        z = jnp.zeros((16,), jnp.float32)
        base = b * _H

        def cstep(i, c):
          a0, a1 = c
          x0, x1 = unpack2(rv[pl.ds(base + i * 16, 16)])
          return a0 + x0, a1 + x1

        a0, a1 = lax.fori_loop(0, _H // 16, cstep, (z, z))
        t0v, t1v = unpack2(rv[pl.ds(base + _H - 16, 16)])
        a0 = a0 + jnp.where(lane >= 8, t0v, 0.0)
        a1 = a1 + jnp.where(lane >= 8, t1v, 0.0)
        o0 = jnp.sum(a0) * inv_h + bias0
        o1 = jnp.sum(a1) * inv_h + bias1
        ov0 = jnp.where(lane == b, o0, ov0)
        ov1 = jnp.where(lane == b, o1, ov1)
      out_v[pl.ds(g * _CROWS, 16)] = ov0
      out_v[pl.ds(_BPW + g * _CROWS, 16)] = ov1

    stage(0)
    fire(jnp.int32(0), rv_a, sem_a)

    def body(h, c):
      g0 = 2 * h
      g1 = 2 * h + 1
      drain(g0, rv_a, sem_a)
      fire(g1, rv_b, sem_b)
      compute(g0, rv_a)
      drain(g1, rv_b, sem_b)
      gn = g1 + 1

      @pl.when(jnp.logical_and(gn < _NCHUNK, gn % 8 == 0))
      def _():
        stage(gn // 8)

      @pl.when(gn < _NCHUNK)
      def _():
        fire(gn, rv_a, sem_a)

      compute(g1, rv_b)
      return c

    lax.fori_loop(0, _NCHUNK // 2, body, 0)
    pltpu.sync_copy(out_v.at[pl.ds(0, _BPW)],
                    out_hbm.at[pl.ds(wid * _BPW, _BPW)])
    pltpu.sync_copy(out_v.at[pl.ds(_BPW, _BPW)],
                    out_hbm.at[pl.ds(_B + wid * _BPW, _BPW)])

  return k


_sc_kernel = _make_sc_kernel()


@jax.jit
def kernel(token_index, emb_table, fc_w, fc_b):
  tok = token_index.astype(jnp.int32).reshape(-1)
  b_pad = jnp.zeros((16,), jnp.float32).at[:2].set(fc_b)
  pp = _project(emb_table.T, fc_w)
  out_flat = _sc_kernel(tok, pp, b_pad)
  return out_flat.reshape(2, _B).T
